# baseline (speedup 1.0000x reference)
import functools

import jax
import jax.numpy as jnp
from jax.experimental import pallas as pl
from jax.experimental.pallas import tpu as pltpu


def _pool_kernel(x_ref, o_ref, *, hw, w_per_row):
    x = x_ref[...]
    bt, g, _ = x.shape
    lane = jax.lax.broadcasted_iota(jnp.int32, (1, 1, 128), 2)
    zero = jnp.zeros((), x.dtype)
    for w in range(w_per_row):
        lo, hi = w * hw, (w + 1) * hw
        parts = []
        for j in range(lo // 128, (hi - 1) // 128 + 1):
            a = max(lo - 128 * j, 0)
            b = min(hi - 128 * j, 128)
            xcol = x[:, :, 128 * j:128 * (j + 1)]
            if a > 0 or b < 128:
                m = (lane >= a) & (lane < b)
                xcol = jnp.where(m, xcol, zero)
            parts.append(jnp.sum(xcol, axis=2, keepdims=True))
        s = parts[0] if len(parts) == 1 else parts[0] + parts[1]
        o_ref[:, w:w + 1, :, :] = s.reshape(bt, 1, g, 1)


def _linear_kernel(p_ref, w_ref, b_ref, o_ref, *, inv_hw):
    acc = jax.lax.dot_general(
        p_ref[...],
        w_ref[...],
        dimension_numbers=(((1,), (1,)), ((), ())),
        preferred_element_type=jnp.float32,
    )
    o_ref[...] = acc * inv_hw + b_ref[...]


def _largest_divisor_at_most(n, cap):
    for d in range(min(n, cap), 0, -1):
        if n % d == 0:
            return d
    return 1


def kernel(x_nchw, weight, bias):
    B, C, H, W = x_nchw.shape
    N = weight.shape[0]
    HW = H * W
    out_dtype = jnp.result_type(x_nchw.dtype, weight.dtype)

    w_per_row = 128
    while C % w_per_row != 0 or (w_per_row * HW) % 128 != 0:
        w_per_row //= 2
        if w_per_row == 1:
            break
    G = C // w_per_row
    x = x_nchw.reshape(B, G, w_per_row * HW)

    bt = _largest_divisor_at_most(B, 8)
    grid = (B // bt,)

    pool_cost = pl.CostEstimate(
        flops=B * C * HW,
        transcendentals=0,
        bytes_accessed=x.size * x.dtype.itemsize + B * C * 4,
    )

    pooled4 = pl.pallas_call(
        functools.partial(_pool_kernel, hw=HW, w_per_row=w_per_row),
        out_shape=jax.ShapeDtypeStruct((B, w_per_row, G, 1), jnp.float32),
        grid=grid,
        in_specs=[pl.BlockSpec((bt, G, w_per_row * HW), lambda i: (i, 0, 0))],
        out_specs=pl.BlockSpec((bt, w_per_row, G, 1), lambda i: (i, 0, 0, 0)),
        compiler_params=pltpu.CompilerParams(
            dimension_semantics=("parallel",),
            vmem_limit_bytes=56 << 20,
        ),
        cost_estimate=pool_cost,
    )(x)

    pooled = jnp.swapaxes(pooled4.reshape(B, w_per_row, G), 1, 2).reshape(B, C)
    bias2 = bias.reshape(1, N)

    bt2 = _largest_divisor_at_most(B, max(1, B // 2))
    lin_grid = (B // bt2,)

    lin_cost = pl.CostEstimate(
        flops=2 * B * C * N,
        transcendentals=0,
        bytes_accessed=B * C * 4 + N * C * weight.dtype.itemsize + B * N * 4,
    )

    scores = pl.pallas_call(
        functools.partial(_linear_kernel, inv_hw=float(1.0 / HW)),
        out_shape=jax.ShapeDtypeStruct((B, N), jnp.float32),
        grid=lin_grid,
        in_specs=[
            pl.BlockSpec((bt2, C), lambda i: (i, 0)),
            pl.BlockSpec((N, C), lambda i: (0, 0)),
            pl.BlockSpec((1, N), lambda i: (0, 0)),
        ],
        out_specs=pl.BlockSpec((bt2, N), lambda i: (i, 0)),
        compiler_params=pltpu.CompilerParams(
            dimension_semantics=("parallel",),
            vmem_limit_bytes=48 << 20,
        ),
        cost_estimate=lin_cost,
    )(pooled, weight, bias2)

    return scores.astype(out_dtype)


# speedup vs baseline: 1.6648x; 1.6648x over previous
import functools

import jax
import jax.numpy as jnp
from jax.experimental import pallas as pl
from jax.experimental.pallas import tpu as pltpu


def _pool_kernel(x_ref, m_ref, o_ref):
    bt, g, l = x_ref.shape
    x2 = x_ref[...].reshape(bt * g, l)
    s = jnp.dot(x2, m_ref[...], preferred_element_type=jnp.float32)
    o_ref[...] = s.reshape(bt, g, m_ref.shape[1])


def _linear_kernel(p_ref, w_ref, b_ref, o_ref, *, inv_hw):
    acc = jax.lax.dot_general(
        p_ref[...],
        w_ref[...],
        dimension_numbers=(((1,), (1,)), ((), ())),
        preferred_element_type=jnp.float32,
    )
    o_ref[...] = acc * inv_hw + b_ref[...]


def _largest_divisor_at_most(n, cap):
    for d in range(min(n, cap), 0, -1):
        if n % d == 0:
            return d
    return 1


def kernel(x_nchw, weight, bias):
    B, C, H, W = x_nchw.shape
    N = weight.shape[0]
    HW = H * W
    out_dtype = jnp.result_type(x_nchw.dtype, weight.dtype)

    w_per_row = 128
    while C % w_per_row != 0 or (w_per_row * HW) % 128 != 0:
        w_per_row //= 2
        if w_per_row == 1:
            break
    G = C // w_per_row
    L = w_per_row * HW
    x = x_nchw.reshape(B, G, L)

    li = jax.lax.broadcasted_iota(jnp.int32, (L, w_per_row), 0)
    wi = jax.lax.broadcasted_iota(jnp.int32, (L, w_per_row), 1)
    m = (li // HW == wi).astype(jnp.float32)

    bt = _largest_divisor_at_most(B, 16)
    grid = (B // bt,)

    pool_cost = pl.CostEstimate(
        flops=2 * B * G * L * w_per_row,
        transcendentals=0,
        bytes_accessed=x.size * x.dtype.itemsize + m.size * 4 + B * C * 4,
    )

    pooled3 = pl.pallas_call(
        _pool_kernel,
        out_shape=jax.ShapeDtypeStruct((B, G, w_per_row), jnp.float32),
        grid=grid,
        in_specs=[
            pl.BlockSpec((bt, G, L), lambda i: (i, 0, 0)),
            pl.BlockSpec((L, w_per_row), lambda i: (0, 0)),
        ],
        out_specs=pl.BlockSpec((bt, G, w_per_row), lambda i: (i, 0, 0)),
        compiler_params=pltpu.CompilerParams(
            dimension_semantics=("parallel",),
            vmem_limit_bytes=56 << 20,
        ),
        cost_estimate=pool_cost,
    )(x, m)

    pooled = pooled3.reshape(B, C)
    bias2 = bias.reshape(1, N)

    bt2 = _largest_divisor_at_most(B, max(1, B // 2))
    lin_grid = (B // bt2,)

    lin_cost = pl.CostEstimate(
        flops=2 * B * C * N,
        transcendentals=0,
        bytes_accessed=B * C * 4 + N * C * weight.dtype.itemsize + B * N * 4,
    )

    scores = pl.pallas_call(
        functools.partial(_linear_kernel, inv_hw=float(1.0 / HW)),
        out_shape=jax.ShapeDtypeStruct((B, N), jnp.float32),
        grid=lin_grid,
        in_specs=[
            pl.BlockSpec((bt2, C), lambda i: (i, 0)),
            pl.BlockSpec((N, C), lambda i: (0, 0)),
            pl.BlockSpec((1, N), lambda i: (0, 0)),
        ],
        out_specs=pl.BlockSpec((bt2, N), lambda i: (i, 0)),
        compiler_params=pltpu.CompilerParams(
            dimension_semantics=("parallel",),
            vmem_limit_bytes=48 << 20,
        ),
        cost_estimate=lin_cost,
    )(pooled, weight, bias2)

    return scores.astype(out_dtype)


# speedup vs baseline: 7.4687x; 4.4863x over previous
import functools

import jax
import jax.numpy as jnp
from jax.experimental import pallas as pl
from jax.experimental.pallas import tpu as pltpu


def _head_kernel(x_ref, w_ref, b_ref, o_ref, *, inv_hw):
    hw = x_ref.shape[0]
    acc = x_ref[0]
    for h in range(1, hw):
        acc = acc + x_ref[h]
    scores = jax.lax.dot_general(
        acc,
        w_ref[...],
        dimension_numbers=(((1,), (0,)), ((), ())),
        preferred_element_type=jnp.float32,
    )
    o_ref[...] = scores * inv_hw + b_ref[...]


def _largest_divisor_at_most(n, cap):
    for d in range(min(n, cap), 0, -1):
        if n % d == 0:
            return d
    return 1


def kernel(x_nchw, weight, bias):
    B, C, H, W = x_nchw.shape
    N = weight.shape[0]
    HW = H * W
    out_dtype = jnp.result_type(x_nchw.dtype, weight.dtype)

    xp = x_nchw.transpose(2, 3, 0, 1).reshape(HW, B, C)
    wp = weight.T
    bias2 = bias.reshape(1, N)

    bt = _largest_divisor_at_most(B, 8)
    grid = (B // bt,)

    cost = pl.CostEstimate(
        flops=B * C * HW + 2 * B * C * N,
        transcendentals=0,
        bytes_accessed=xp.size * 4 + C * N * 4 + B * N * 4,
    )

    scores = pl.pallas_call(
        functools.partial(_head_kernel, inv_hw=float(1.0 / HW)),
        out_shape=jax.ShapeDtypeStruct((B, N), jnp.float32),
        grid=grid,
        in_specs=[
            pl.BlockSpec((HW, bt, C), lambda i: (0, i, 0)),
            pl.BlockSpec((C, N), lambda i: (0, 0)),
            pl.BlockSpec((1, N), lambda i: (0, 0)),
        ],
        out_specs=pl.BlockSpec((bt, N), lambda i: (i, 0)),
        compiler_params=pltpu.CompilerParams(
            dimension_semantics=("parallel",),
            vmem_limit_bytes=48 << 20,
        ),
        cost_estimate=cost,
    )(xp, wp, bias2)

    return scores.astype(out_dtype)


# speedup vs baseline: 14.6476x; 1.9612x over previous
import functools

import jax
import jax.numpy as jnp
from jax.experimental import pallas as pl
from jax.experimental.pallas import tpu as pltpu


def _head_kernel(x_ref, w_ref, b_ref, o_ref, *, inv_hw):
    hw = x_ref.shape[0]
    acc = x_ref[0]
    for h in range(1, hw):
        acc = acc + x_ref[h]
    scores = jax.lax.dot_general(
        acc,
        w_ref[...],
        dimension_numbers=(((1,), (1,)), ((), ())),
        preferred_element_type=jnp.float32,
    )
    o_ref[...] = scores * inv_hw + b_ref[...]


def _largest_divisor_at_most(n, cap):
    for d in range(min(n, cap), 0, -1):
        if n % d == 0:
            return d
    return 1


def kernel(x_nchw, weight, bias):
    B, C, H, W = x_nchw.shape
    N = weight.shape[0]
    HW = H * W
    out_dtype = jnp.result_type(x_nchw.dtype, weight.dtype)

    xp = x_nchw.transpose(2, 3, 0, 1).reshape(HW, B, C)
    bias2 = bias.reshape(1, N)

    bt = _largest_divisor_at_most(B, 32)
    grid = (B // bt,)

    cost = pl.CostEstimate(
        flops=B * C * HW + 2 * B * C * N,
        transcendentals=0,
        bytes_accessed=xp.size * 4 + C * N * 4 + B * N * 4,
    )

    scores = pl.pallas_call(
        functools.partial(_head_kernel, inv_hw=float(1.0 / HW)),
        out_shape=jax.ShapeDtypeStruct((B, N), jnp.float32),
        grid=grid,
        in_specs=[
            pl.BlockSpec((HW, bt, C), lambda i: (0, i, 0)),
            pl.BlockSpec((N, C), lambda i: (0, 0)),
            pl.BlockSpec((1, N), lambda i: (0, 0)),
        ],
        out_specs=pl.BlockSpec((bt, N), lambda i: (i, 0)),
        compiler_params=pltpu.CompilerParams(
            dimension_semantics=("parallel",),
            vmem_limit_bytes=48 << 20,
        ),
        cost_estimate=cost,
    )(xp, weight, bias2)

    return scores.astype(out_dtype)
